# baseline (speedup 1.0000x reference)
import functools

import jax
import jax.numpy as jnp
from jax import lax
from jax.experimental import pallas as pl
from jax.experimental.pallas import tpu as pltpu
from jax.experimental.pallas import tpu_sc as plsc

N = 10000
D = 128
H = 64
B = 1024

NC = 2
NS = 16
NW = NC * NS
LANE = 128

NACC1 = 10240
RPT1 = NACC1 // NS
NACC2 = 10240
OUT_RPT = NACC2 // NS

_mesh = plsc.VectorSubcoreMesh(core_axis_name="c", subcore_axis_name="s")
_SC_PARAMS = pltpu.CompilerParams(use_tc_tiling_on_sc=False)


def _worker_counts(num_rows):
  rpw = (num_rows + NW - 1) // NW
  return (rpw + 7) // 8 * 8


def _make_deg(rpad, rreal, rpw):
  @functools.partial(
      pl.kernel,
      mesh=_mesh,
      compiler_params=_SC_PARAMS,
      out_type=jax.ShapeDtypeStruct((NC, 2, NACC1), jnp.float32),
      scratch_types=[
          pltpu.VMEM_SHARED((NACC1,), jnp.float32),
          pltpu.VMEM_SHARED((NACC1,), jnp.float32),
          pltpu.VMEM((LANE,), jnp.float32),
          pltpu.VMEM((rpw, LANE), jnp.int32),
          pltpu.VMEM((rpw, LANE), jnp.int32),
          pltpu.SemaphoreType.DMA,
      ],
  )
  def deg_kernel(srcr, dstr, zeros1, out, acc_o, acc_i, ones_v, src_v, dst_v,
                 sem):
    c = lax.axis_index("c")
    s = lax.axis_index("s")
    w = s * NC + c
    pltpu.sync_copy(zeros1.at[pl.ds(s * RPT1, RPT1)],
                    acc_o.at[pl.ds(s * RPT1, RPT1)])
    pltpu.sync_copy(zeros1.at[pl.ds(s * RPT1, RPT1)],
                    acc_i.at[pl.ds(s * RPT1, RPT1)])

    def fill(i, carry):
      ones_v[pl.ds(i * 16, 16)] = jnp.full((16,), 1.0, jnp.float32)
      return carry

    lax.fori_loop(0, LANE // 16, fill, 0)
    pltpu.sync_copy(srcr.at[pl.ds(w * rpw, rpw)], src_v)
    pltpu.sync_copy(dstr.at[pl.ds(w * rpw, rpw)], dst_v)
    plsc.subcore_barrier()

    K = 8

    def prime(j, carry):
      pltpu.async_copy(ones_v, acc_o.at[src_v.at[j]], sem, add=True)
      pltpu.async_copy(ones_v, acc_i.at[dst_v.at[j]], sem, add=True)
      return carry

    lax.fori_loop(0, K, prime, 0)

    def step(j, carry):
      pltpu.async_copy(ones_v, acc_o.at[src_v.at[j]], sem, add=True)
      pltpu.async_copy(ones_v, acc_i.at[dst_v.at[j]], sem, add=True)
      pltpu.make_async_copy(ones_v, acc_o.at[src_v.at[j - K]], sem).wait()
      pltpu.make_async_copy(ones_v, acc_i.at[dst_v.at[j - K]], sem).wait()
      return carry

    lax.fori_loop(K, rpw, step, 0)

    def tail(j, carry):
      pltpu.make_async_copy(ones_v, acc_o.at[src_v.at[j]], sem).wait()
      pltpu.make_async_copy(ones_v, acc_i.at[dst_v.at[j]], sem).wait()
      return carry

    lax.fori_loop(rpw - K, rpw, tail, 0)
    plsc.subcore_barrier()
    pltpu.sync_copy(acc_o.at[pl.ds(s * RPT1, RPT1)],
                    out.at[c, 0, pl.ds(s * RPT1, RPT1)])
    pltpu.sync_copy(acc_i.at[pl.ds(s * RPT1, RPT1)],
                    out.at[c, 1, pl.ds(s * RPT1, RPT1)])

  return deg_kernel


def _make_seg(rpad, rreal, rpw):
  @functools.partial(
      pl.kernel,
      mesh=_mesh,
      compiler_params=_SC_PARAMS,
      out_type=(jax.ShapeDtypeStruct((NACC2, H), jnp.float32),
                jax.ShapeDtypeStruct((NACC2, H), jnp.float32)),
      scratch_types=[
          pltpu.VMEM_SHARED((NACC2, H), jnp.float32),
          pltpu.VMEM((LANE, H), jnp.float32),
          pltpu.VMEM((LANE, H), jnp.float32),
          pltpu.VMEM((LANE, H), jnp.float32),
          pltpu.VMEM((LANE, H), jnp.float32),
          pltpu.VMEM((LANE, H), jnp.float32),
          pltpu.VMEM((LANE, H), jnp.float32),
          pltpu.VMEM((LANE, H), jnp.float32),
          pltpu.VMEM((LANE, H), jnp.float32),
          pltpu.VMEM((rpw, LANE), jnp.int32),
          pltpu.VMEM((rpw, LANE), jnp.int32),
          pltpu.SemaphoreType.DMA,
          pltpu.SemaphoreType.DMA,
          pltpu.SemaphoreType.DMA,
          pltpu.SemaphoreType.DMA,
          pltpu.SemaphoreType.DMA,
          pltpu.SemaphoreType.DMA,
          pltpu.SemaphoreType.DMA,
          pltpu.SemaphoreType.DMA,
          pltpu.SemaphoreType.DMA,
          pltpu.SemaphoreType.DMA,
          pltpu.SemaphoreType.DMA,
          pltpu.SemaphoreType.DMA,
          pltpu.SemaphoreType.DMA,
          pltpu.SemaphoreType.DMA,
          pltpu.SemaphoreType.DMA,
          pltpu.SemaphoreType.DMA,
      ],
  )
  def seg_kernel(t, srcr, dstr, zeros2, out0, out1, acc, rows_0, rows_1,
                 rows_2, rows_3, rows_4, rows_5, rows_6, rows_7, src_v,
                 dst_v, g0, g1, g2, g3, g4, g5, g6, g7, s0, s1, s2, s3, s4,
                 s5, s6, s7):
    c = lax.axis_index("c")
    s = lax.axis_index("s")
    w = s * NC + c
    pltpu.sync_copy(zeros2.at[pl.ds(s * OUT_RPT, OUT_RPT)],
                    acc.at[pl.ds(s * OUT_RPT, OUT_RPT)])
    pltpu.sync_copy(srcr.at[pl.ds(w * rpw, rpw)], src_v)
    pltpu.sync_copy(dstr.at[pl.ds(w * rpw, rpw)], dst_v)
    plsc.subcore_barrier()

    bufs = ((rows_0, g0, s0), (rows_1, g1, s1),
            (rows_2, g2, s2), (rows_3, g3, s3),
            (rows_4, g4, s4), (rows_5, g5, s5),
            (rows_6, g6, s6), (rows_7, g7, s7))
    noct = rpw // 8

    for k, (buf, gsem, _) in enumerate(bufs):
      pltpu.async_copy(t.at[src_v.at[k]], buf, gsem)

    def octet(i, carry):
      j = 8 * i
      for k, (buf, gsem, ssem) in enumerate(bufs):
        pltpu.make_async_copy(t.at[src_v.at[j + k]], buf, gsem).wait()
        pltpu.async_copy(buf, acc.at[dst_v.at[j + k]], ssem, add=True)
      for k, (buf, gsem, ssem) in enumerate(bufs):
        pltpu.make_async_copy(buf, acc.at[dst_v.at[j + k]], ssem).wait()
        pltpu.async_copy(t.at[src_v.at[j + 8 + k]], buf, gsem)
      return carry

    lax.fori_loop(0, noct - 1, octet, 0)

    jl = 8 * (noct - 1)
    for k, (buf, gsem, ssem) in enumerate(bufs):
      pltpu.make_async_copy(t.at[src_v.at[jl + k]], buf, gsem).wait()
      pltpu.async_copy(buf, acc.at[dst_v.at[jl + k]], ssem, add=True)
    for k, (buf, gsem, ssem) in enumerate(bufs):
      pltpu.make_async_copy(buf, acc.at[dst_v.at[jl + k]], ssem).wait()

    plsc.subcore_barrier()

    @pl.when(c == 0)
    def _():
      pltpu.sync_copy(acc.at[pl.ds(s * OUT_RPT, OUT_RPT)],
                      out0.at[pl.ds(s * OUT_RPT, OUT_RPT)])

    @pl.when(c == 1)
    def _():
      pltpu.sync_copy(acc.at[pl.ds(s * OUT_RPT, OUT_RPT)],
                      out1.at[pl.ds(s * OUT_RPT, OUT_RPT)])

  return seg_kernel


def _make_seg_gather(rpad, rreal, rpw):
  @functools.partial(
      pl.kernel,
      mesh=_mesh,
      compiler_params=_SC_PARAMS,
      out_type=(jax.ShapeDtypeStruct((2 * B, H), jnp.float32),
                jax.ShapeDtypeStruct((2 * B, H), jnp.float32),
                jax.ShapeDtypeStruct((2 * B, 16), jnp.float32)),
      scratch_types=[
          pltpu.VMEM_SHARED((NACC2, H), jnp.float32),
          pltpu.VMEM((LANE, H), jnp.float32),
          pltpu.VMEM((LANE, H), jnp.float32),
          pltpu.VMEM((LANE, H), jnp.float32),
          pltpu.VMEM((LANE, H), jnp.float32),
          pltpu.VMEM((LANE, H), jnp.float32),
          pltpu.VMEM((LANE, H), jnp.float32),
          pltpu.VMEM((LANE, H), jnp.float32),
          pltpu.VMEM((LANE, H), jnp.float32),
          pltpu.VMEM((LANE, 16), jnp.float32),
          pltpu.VMEM((1, LANE), jnp.int32),
          pltpu.VMEM((rpw, LANE), jnp.int32),
          pltpu.VMEM((rpw, LANE), jnp.int32),
          pltpu.SemaphoreType.DMA,
          pltpu.SemaphoreType.DMA,
          pltpu.SemaphoreType.DMA,
          pltpu.SemaphoreType.DMA,
          pltpu.SemaphoreType.DMA,
          pltpu.SemaphoreType.DMA,
          pltpu.SemaphoreType.DMA,
          pltpu.SemaphoreType.DMA,
          pltpu.SemaphoreType.DMA,
          pltpu.SemaphoreType.DMA,
          pltpu.SemaphoreType.DMA,
          pltpu.SemaphoreType.DMA,
          pltpu.SemaphoreType.DMA,
          pltpu.SemaphoreType.DMA,
          pltpu.SemaphoreType.DMA,
          pltpu.SemaphoreType.DMA,
      ],
  )
  def seg_gather_kernel(t, srcr, dstr, zeros2, nib, tir, c0_out, c1_out,
                        nit_out, acc, rows_0, rows_1, rows_2, rows_3, rows_4,
                        rows_5, rows_6, rows_7, niv, idx_v, src_v, dst_v, g0,
                        g1, g2, g3, g4, g5, g6, g7, s0, s1, s2, s3, s4, s5,
                        s6, s7):
    c = lax.axis_index("c")
    s = lax.axis_index("s")
    w = s * NC + c
    pltpu.sync_copy(zeros2.at[pl.ds(s * OUT_RPT, OUT_RPT)],
                    acc.at[pl.ds(s * OUT_RPT, OUT_RPT)])
    pltpu.sync_copy(srcr.at[pl.ds(w * rpw, rpw)], src_v)
    pltpu.sync_copy(dstr.at[pl.ds(w * rpw, rpw)], dst_v)
    plsc.subcore_barrier()

    bufs = ((rows_0, g0, s0), (rows_1, g1, s1),
            (rows_2, g2, s2), (rows_3, g3, s3),
            (rows_4, g4, s4), (rows_5, g5, s5),
            (rows_6, g6, s6), (rows_7, g7, s7))
    noct = rpw // 8

    for k, (buf, gsem, _) in enumerate(bufs):
      pltpu.async_copy(t.at[src_v.at[k]], buf, gsem)

    def octet(i, carry):
      j = 8 * i
      for k, (buf, gsem, ssem) in enumerate(bufs):
        pltpu.make_async_copy(t.at[src_v.at[j + k]], buf, gsem).wait()
        pltpu.async_copy(buf, acc.at[dst_v.at[j + k]], ssem, add=True)
      for k, (buf, gsem, ssem) in enumerate(bufs):
        pltpu.make_async_copy(buf, acc.at[dst_v.at[j + k]], ssem).wait()
        pltpu.async_copy(t.at[src_v.at[j + 8 + k]], buf, gsem)
      return carry

    lax.fori_loop(0, noct - 1, octet, 0)

    jl = 8 * (noct - 1)
    for k, (buf, gsem, ssem) in enumerate(bufs):
      pltpu.make_async_copy(t.at[src_v.at[jl + k]], buf, gsem).wait()
      pltpu.async_copy(buf, acc.at[dst_v.at[jl + k]], ssem, add=True)
    for k, (buf, gsem, ssem) in enumerate(bufs):
      pltpu.make_async_copy(buf, acc.at[dst_v.at[jl + k]], ssem).wait()

    plsc.subcore_barrier()

    pltpu.sync_copy(tir.at[s], idx_v)
    pltpu.async_copy(acc.at[idx_v.at[0]], rows_0, g0).wait()

    @pl.when(c == 0)
    def _():
      pltpu.sync_copy(rows_0, c0_out.at[pl.ds(s * LANE, LANE)])
      pltpu.async_copy(nib.at[idx_v.at[0]], niv, g1).wait()
      pltpu.sync_copy(niv, nit_out.at[pl.ds(s * LANE, LANE)])

    @pl.when(c == 1)
    def _():
      pltpu.sync_copy(rows_0, c1_out.at[pl.ds(s * LANE, LANE)])

  return seg_gather_kernel


@functools.partial(
    pl.kernel,
    mesh=_mesh,
    compiler_params=_SC_PARAMS,
    out_type=(jax.ShapeDtypeStruct((2 * B, H), jnp.float32),
              jax.ShapeDtypeStruct((2 * B, H), jnp.float32),
              jax.ShapeDtypeStruct((2 * B, 16), jnp.float32)),
    scratch_types=[
        pltpu.VMEM((LANE, H), jnp.float32),
        pltpu.VMEM((LANE, H), jnp.float32),
        pltpu.VMEM((LANE, 16), jnp.float32),
        pltpu.VMEM((1, LANE), jnp.int32),
        pltpu.SemaphoreType.DMA,
    ],
)
def _team_gather(q0, q1, nib, tir, out0, out1, out_ni, rows0, rows1, niv,
                 idx_v, sem):
  c = lax.axis_index("c")
  s = lax.axis_index("s")
  w = s * NC + c
  nrows = (2 * B) // LANE

  @pl.when(w < nrows)
  def _():
    pltpu.sync_copy(tir.at[w], idx_v)
    pltpu.async_copy(q0.at[idx_v.at[0]], rows0, sem)
    pltpu.async_copy(q1.at[idx_v.at[0]], rows1, sem)
    pltpu.async_copy(nib.at[idx_v.at[0]], niv, sem)
    pltpu.make_async_copy(q0.at[idx_v.at[0]], rows0, sem).wait()
    pltpu.make_async_copy(q1.at[idx_v.at[0]], rows1, sem).wait()
    pltpu.make_async_copy(nib.at[idx_v.at[0]], niv, sem).wait()
    pltpu.sync_copy(rows0, out0.at[pl.ds(w * LANE, LANE)])
    pltpu.sync_copy(rows1, out1.at[pl.ds(w * LANE, LANE)])
    pltpu.sync_copy(niv, out_ni.at[pl.ds(w * LANE, LANE)])


_BN = 2000


def _prep_body(h_ref, dego_ref, degi_ref, w1_ref, t1_ref, no_ref, ni_ref,
               nib_ref):
  do = dego_ref[:, 0:1] + dego_ref[:, 1:2]
  di = degi_ref[:, 0:1] + degi_ref[:, 1:2]
  no = lax.rsqrt(jnp.maximum(do, 1.0))
  ni = lax.rsqrt(jnp.maximum(di, 1.0))
  no_ref[...] = no
  ni_ref[...] = ni
  nib_ref[...] = jnp.broadcast_to(ni, (ni.shape[0], 16))
  t1_ref[...] = jnp.dot(h_ref[...] * no, w1_ref[...],
                        preferred_element_type=jnp.float32)


def _prep(h, dego, degi, w1):
  grid = (N // _BN,)
  return pl.pallas_call(
      _prep_body,
      grid=grid,
      in_specs=[
          pl.BlockSpec((_BN, D), lambda i: (i, 0)),
          pl.BlockSpec((_BN, 2), lambda i: (i, 0)),
          pl.BlockSpec((_BN, 2), lambda i: (i, 0)),
          pl.BlockSpec((D, H), lambda i: (0, 0)),
      ],
      out_specs=[
          pl.BlockSpec((_BN, H), lambda i: (i, 0)),
          pl.BlockSpec((_BN, 1), lambda i: (i, 0)),
          pl.BlockSpec((_BN, 1), lambda i: (i, 0)),
          pl.BlockSpec((_BN, 16), lambda i: (i, 0)),
      ],
      out_shape=[
          jax.ShapeDtypeStruct((N, H), jnp.float32),
          jax.ShapeDtypeStruct((N, 1), jnp.float32),
          jax.ShapeDtypeStruct((N, 1), jnp.float32),
          jax.ShapeDtypeStruct((N, 16), jnp.float32),
      ],
  )(h, dego, degi, w1)


def _mid_body(p0_ref, p1_ref, ni_ref, no_ref, w2_ref, b1_ref, t2_ref):
  x = jnp.maximum((p0_ref[...] + p1_ref[...]) * ni_ref[...] + b1_ref[...],
                  0.0)
  t2_ref[...] = jnp.dot(x * no_ref[...], w2_ref[...],
                        preferred_element_type=jnp.float32)


def _mid(p0, p1, ni, no, w2, b1):
  grid = (N // _BN,)
  return pl.pallas_call(
      _mid_body,
      grid=grid,
      in_specs=[
          pl.BlockSpec((_BN, H), lambda i: (i, 0)),
          pl.BlockSpec((_BN, H), lambda i: (i, 0)),
          pl.BlockSpec((_BN, 1), lambda i: (i, 0)),
          pl.BlockSpec((_BN, 1), lambda i: (i, 0)),
          pl.BlockSpec((H, H), lambda i: (0, 0)),
          pl.BlockSpec((1, H), lambda i: (0, 0)),
      ],
      out_specs=pl.BlockSpec((_BN, H), lambda i: (i, 0)),
      out_shape=jax.ShapeDtypeStruct((N, H), jnp.float32),
  )(p0, p1, ni, no, w2, b1)


def _head_body(c0_ref, c1_ref, nit_ref, b2_ref, wp1_ref, bp1_ref, wp2_ref,
               bp2_ref, out_ref):
  x2 = (c0_ref[...] + c1_ref[...]) * nit_ref[:, 0:1] + b2_ref[...]
  home = x2[0:B]
  away = x2[B:2 * B]
  z = (jnp.dot(home, wp1_ref[0:H], preferred_element_type=jnp.float32)
       + jnp.dot(away, wp1_ref[H:2 * H], preferred_element_type=jnp.float32)
       + bp1_ref[...])
  z = jnp.maximum(z, 0.0)
  out_ref[...] = jnp.dot(z, wp2_ref[...],
                         preferred_element_type=jnp.float32) + bp2_ref[...]


def _head(c0, c1, nit, b2, wp1, bp1, wp2, bp2):
  return pl.pallas_call(
      _head_body,
      out_shape=jax.ShapeDtypeStruct((B, 2), jnp.float32),
  )(c0, c1, nit, b2, wp1, bp1, wp2, bp2)


def kernel(h, edge_index, team_indices, W1, b1, W2, b2, Wp1, bp1, Wp2, bp2):
  e = edge_index.shape[1]
  rreal = e // LANE
  rpw = _worker_counts(rreal)
  rpad = rpw * NW

  src = edge_index[0].astype(jnp.int32)
  dst = edge_index[1].astype(jnp.int32)
  pad = rpad * LANE - e
  srcr = jnp.concatenate([src, jnp.zeros((pad,), jnp.int32)]).reshape(rpad, LANE)
  srcr_deg = jnp.concatenate([src, jnp.full((pad,), N, jnp.int32)]).reshape(rpad, LANE)
  dstr = jnp.concatenate([dst, jnp.full((pad,), N, jnp.int32)]).reshape(rpad, LANE)

  zeros1 = jnp.zeros((NACC1,), jnp.float32)
  zeros2 = jnp.zeros((NACC2, H), jnp.float32)

  degp = _make_deg(rpad, rreal, rpw)(srcr_deg, dstr, zeros1)
  dego = degp[:, 0, :N].T
  degi = degp[:, 1, :N].T

  t1, no, ni, nib = _prep(h, dego, degi, W1)
  seg = _make_seg(rpad, rreal, rpw)
  p0, p1 = seg(t1, srcr, dstr, zeros2)
  t2 = _mid(p0, p1, ni, no, W2, b1.reshape(1, H))
  tir = team_indices.T.astype(jnp.int32).reshape(2 * B // LANE, 1, LANE)
  c0, c1, nit = _make_seg_gather(rpad, rreal, rpw)(t2, srcr, dstr, zeros2,
                                                   nib, tir)

  return _head(c0, c1, nit, b2.reshape(1, H), Wp1,
               bp1.reshape(1, H), Wp2, bp2.reshape(1, 2))


# speedup vs baseline: 2.5265x; 2.5265x over previous
import functools

import jax
import jax.numpy as jnp
from jax import lax
from jax.experimental import pallas as pl
from jax.experimental.pallas import tpu as pltpu
from jax.experimental.pallas import tpu_sc as plsc

N = 10000
D = 128
H = 64
B = 1024

NC = 2
NS = 16
NW = NC * NS
LANE = 128

NACC1 = 10240
RPT1 = NACC1 // NS
NACC2 = 10240
OUT_RPT = NACC2 // NS

_mesh = plsc.VectorSubcoreMesh(core_axis_name="c", subcore_axis_name="s")
_SC_PARAMS = pltpu.CompilerParams(use_tc_tiling_on_sc=False)


def _worker_counts(num_rows):
  rpw = (num_rows + NW - 1) // NW
  return (rpw + 7) // 8 * 8


def _make_deg(rpad, rreal, rpw):
  @functools.partial(
      pl.kernel,
      mesh=_mesh,
      compiler_params=_SC_PARAMS,
      out_type=jax.ShapeDtypeStruct((NC, 2, NACC1), jnp.float32),
      scratch_types=[
          pltpu.VMEM_SHARED((NACC1,), jnp.float32),
          pltpu.VMEM_SHARED((NACC1,), jnp.float32),
          pltpu.VMEM((LANE,), jnp.float32),
          pltpu.VMEM((rpw, LANE), jnp.int32),
          pltpu.VMEM((rpw, LANE), jnp.int32),
          pltpu.SemaphoreType.DMA,
      ],
  )
  def deg_kernel(srcr, dstr, zeros1, out, acc_o, acc_i, ones_v, src_v, dst_v,
                 sem):
    c = lax.axis_index("c")
    s = lax.axis_index("s")
    w = s * NC + c
    pltpu.sync_copy(zeros1.at[pl.ds(s * RPT1, RPT1)],
                    acc_o.at[pl.ds(s * RPT1, RPT1)])
    pltpu.sync_copy(zeros1.at[pl.ds(s * RPT1, RPT1)],
                    acc_i.at[pl.ds(s * RPT1, RPT1)])

    def fill(i, carry):
      ones_v[pl.ds(i * 16, 16)] = jnp.full((16,), 1.0, jnp.float32)
      return carry

    lax.fori_loop(0, LANE // 16, fill, 0)
    pltpu.sync_copy(srcr.at[pl.ds(w * rpw, rpw)], src_v)
    pltpu.sync_copy(dstr.at[pl.ds(w * rpw, rpw)], dst_v)
    plsc.subcore_barrier()

    cnt = jnp.minimum(rpw, jnp.maximum(0, rreal - w * rpw))

    K = 8

    def step(j, carry):
      pltpu.async_copy(ones_v, acc_o.at[src_v.at[j]], sem, add=True)
      pltpu.async_copy(ones_v, acc_i.at[dst_v.at[j]], sem, add=True)

      @pl.when(j >= K)
      def _():
        pltpu.make_async_copy(ones_v, acc_o.at[src_v.at[j - K]], sem).wait()
        pltpu.make_async_copy(ones_v, acc_i.at[dst_v.at[j - K]], sem).wait()

      return carry

    lax.fori_loop(0, cnt, step, 0)

    def tail(j, carry):
      pltpu.make_async_copy(ones_v, acc_o.at[src_v.at[j]], sem).wait()
      pltpu.make_async_copy(ones_v, acc_i.at[dst_v.at[j]], sem).wait()
      return carry

    lax.fori_loop(jnp.maximum(0, cnt - K), cnt, tail, 0)
    plsc.subcore_barrier()
    pltpu.sync_copy(acc_o.at[pl.ds(s * RPT1, RPT1)],
                    out.at[c, 0, pl.ds(s * RPT1, RPT1)])
    pltpu.sync_copy(acc_i.at[pl.ds(s * RPT1, RPT1)],
                    out.at[c, 1, pl.ds(s * RPT1, RPT1)])

  return deg_kernel


def _make_seg(rpad, rreal, rpw):
  @functools.partial(
      pl.kernel,
      mesh=_mesh,
      compiler_params=_SC_PARAMS,
      out_type=(jax.ShapeDtypeStruct((NACC2, H), jnp.float32),
                jax.ShapeDtypeStruct((NACC2, H), jnp.float32)),
      scratch_types=[
          pltpu.VMEM_SHARED((NACC2, H), jnp.float32),
          pltpu.VMEM((LANE, H), jnp.float32),
          pltpu.VMEM((LANE, H), jnp.float32),
          pltpu.VMEM((LANE, H), jnp.float32),
          pltpu.VMEM((LANE, H), jnp.float32),
          pltpu.VMEM((rpw, LANE), jnp.int32),
          pltpu.VMEM((rpw, LANE), jnp.int32),
          pltpu.SemaphoreType.DMA,
          pltpu.SemaphoreType.DMA,
          pltpu.SemaphoreType.DMA,
          pltpu.SemaphoreType.DMA,
          pltpu.SemaphoreType.DMA,
          pltpu.SemaphoreType.DMA,
          pltpu.SemaphoreType.DMA,
          pltpu.SemaphoreType.DMA,
      ],
  )
  def seg_kernel(t, srcr, dstr, zeros2, out0, out1, acc, rows_0, rows_1,
                 rows_2, rows_3, src_v, dst_v, g0, g1, g2, g3, s0, s1, s2,
                 s3):
    c = lax.axis_index("c")
    s = lax.axis_index("s")
    w = s * NC + c
    pltpu.sync_copy(zeros2.at[pl.ds(s * OUT_RPT, OUT_RPT)],
                    acc.at[pl.ds(s * OUT_RPT, OUT_RPT)])
    pltpu.sync_copy(srcr.at[pl.ds(w * rpw, rpw)], src_v)
    pltpu.sync_copy(dstr.at[pl.ds(w * rpw, rpw)], dst_v)
    plsc.subcore_barrier()

    cnt = jnp.minimum(rpw, jnp.maximum(0, rreal - w * rpw))
    bufs = ((rows_0, g0, s0), (rows_1, g1, s1),
            (rows_2, g2, s2), (rows_3, g3, s3))

    for k, (buf, gsem, _) in enumerate(bufs):
      @pl.when(k < cnt)
      def _(k=k, buf=buf, gsem=gsem):
        pltpu.async_copy(t.at[src_v.at[k]], buf, gsem)

    def quad(i, carry):
      j = 4 * i
      for k, (buf, gsem, ssem) in enumerate(bufs):
        pltpu.make_async_copy(t.at[src_v.at[j + k]], buf, gsem).wait()
        pltpu.async_copy(buf, acc.at[dst_v.at[j + k]], ssem, add=True)
      for k, (buf, gsem, ssem) in enumerate(bufs):
        pltpu.make_async_copy(buf, acc.at[dst_v.at[j + k]], ssem).wait()

        @pl.when(j + 4 + k < cnt)
        def _(k=k, buf=buf, gsem=gsem, j=j):
          pltpu.async_copy(t.at[src_v.at[j + 4 + k]], buf, gsem)

      return carry

    lax.fori_loop(0, cnt // 4, quad, 0)
    plsc.subcore_barrier()

    @pl.when(c == 0)
    def _():
      pltpu.sync_copy(acc.at[pl.ds(s * OUT_RPT, OUT_RPT)],
                      out0.at[pl.ds(s * OUT_RPT, OUT_RPT)])

    @pl.when(c == 1)
    def _():
      pltpu.sync_copy(acc.at[pl.ds(s * OUT_RPT, OUT_RPT)],
                      out1.at[pl.ds(s * OUT_RPT, OUT_RPT)])

  return seg_kernel


def _make_seg_gather(rpad, rreal, rpw):
  @functools.partial(
      pl.kernel,
      mesh=_mesh,
      compiler_params=_SC_PARAMS,
      out_type=(jax.ShapeDtypeStruct((2 * B, H), jnp.float32),
                jax.ShapeDtypeStruct((2 * B, H), jnp.float32),
                jax.ShapeDtypeStruct((2 * B, 16), jnp.float32)),
      scratch_types=[
          pltpu.VMEM_SHARED((NACC2, H), jnp.float32),
          pltpu.VMEM((LANE, H), jnp.float32),
          pltpu.VMEM((LANE, H), jnp.float32),
          pltpu.VMEM((LANE, H), jnp.float32),
          pltpu.VMEM((LANE, H), jnp.float32),
          pltpu.VMEM((LANE, 16), jnp.float32),
          pltpu.VMEM((1, LANE), jnp.int32),
          pltpu.VMEM((rpw, LANE), jnp.int32),
          pltpu.VMEM((rpw, LANE), jnp.int32),
          pltpu.SemaphoreType.DMA,
          pltpu.SemaphoreType.DMA,
          pltpu.SemaphoreType.DMA,
          pltpu.SemaphoreType.DMA,
          pltpu.SemaphoreType.DMA,
          pltpu.SemaphoreType.DMA,
          pltpu.SemaphoreType.DMA,
          pltpu.SemaphoreType.DMA,
      ],
  )
  def seg_gather_kernel(t, srcr, dstr, zeros2, nib, tir, c0_out, c1_out,
                        nit_out, acc, rows_0, rows_1, rows_2, rows_3, niv,
                        idx_v, src_v, dst_v, g0, g1, g2, g3, s0, s1, s2, s3):
    c = lax.axis_index("c")
    s = lax.axis_index("s")
    w = s * NC + c
    pltpu.sync_copy(zeros2.at[pl.ds(s * OUT_RPT, OUT_RPT)],
                    acc.at[pl.ds(s * OUT_RPT, OUT_RPT)])
    pltpu.sync_copy(srcr.at[pl.ds(w * rpw, rpw)], src_v)
    pltpu.sync_copy(dstr.at[pl.ds(w * rpw, rpw)], dst_v)
    plsc.subcore_barrier()

    cnt = jnp.minimum(rpw, jnp.maximum(0, rreal - w * rpw))
    bufs = ((rows_0, g0, s0), (rows_1, g1, s1),
            (rows_2, g2, s2), (rows_3, g3, s3))

    for k, (buf, gsem, _) in enumerate(bufs):
      @pl.when(k < cnt)
      def _(k=k, buf=buf, gsem=gsem):
        pltpu.async_copy(t.at[src_v.at[k]], buf, gsem)

    def quad(i, carry):
      j = 4 * i
      for k, (buf, gsem, ssem) in enumerate(bufs):
        pltpu.make_async_copy(t.at[src_v.at[j + k]], buf, gsem).wait()
        pltpu.async_copy(buf, acc.at[dst_v.at[j + k]], ssem, add=True)
      for k, (buf, gsem, ssem) in enumerate(bufs):
        pltpu.make_async_copy(buf, acc.at[dst_v.at[j + k]], ssem).wait()

        @pl.when(j + 4 + k < cnt)
        def _(k=k, buf=buf, gsem=gsem, j=j):
          pltpu.async_copy(t.at[src_v.at[j + 4 + k]], buf, gsem)

      return carry

    lax.fori_loop(0, cnt // 4, quad, 0)
    plsc.subcore_barrier()

    pltpu.sync_copy(tir.at[s], idx_v)
    pltpu.async_copy(acc.at[idx_v.at[0]], rows_0, g0).wait()

    @pl.when(c == 0)
    def _():
      pltpu.sync_copy(rows_0, c0_out.at[pl.ds(s * LANE, LANE)])
      pltpu.async_copy(nib.at[idx_v.at[0]], niv, g1).wait()
      pltpu.sync_copy(niv, nit_out.at[pl.ds(s * LANE, LANE)])

    @pl.when(c == 1)
    def _():
      pltpu.sync_copy(rows_0, c1_out.at[pl.ds(s * LANE, LANE)])

  return seg_gather_kernel


@functools.partial(
    pl.kernel,
    mesh=_mesh,
    compiler_params=_SC_PARAMS,
    out_type=(jax.ShapeDtypeStruct((2 * B, H), jnp.float32),
              jax.ShapeDtypeStruct((2 * B, H), jnp.float32),
              jax.ShapeDtypeStruct((2 * B, 16), jnp.float32)),
    scratch_types=[
        pltpu.VMEM((LANE, H), jnp.float32),
        pltpu.VMEM((LANE, H), jnp.float32),
        pltpu.VMEM((LANE, 16), jnp.float32),
        pltpu.VMEM((1, LANE), jnp.int32),
        pltpu.SemaphoreType.DMA,
    ],
)
def _team_gather(q0, q1, nib, tir, out0, out1, out_ni, rows0, rows1, niv,
                 idx_v, sem):
  c = lax.axis_index("c")
  s = lax.axis_index("s")
  w = s * NC + c
  nrows = (2 * B) // LANE

  @pl.when(w < nrows)
  def _():
    pltpu.sync_copy(tir.at[w], idx_v)
    pltpu.async_copy(q0.at[idx_v.at[0]], rows0, sem)
    pltpu.async_copy(q1.at[idx_v.at[0]], rows1, sem)
    pltpu.async_copy(nib.at[idx_v.at[0]], niv, sem)
    pltpu.make_async_copy(q0.at[idx_v.at[0]], rows0, sem).wait()
    pltpu.make_async_copy(q1.at[idx_v.at[0]], rows1, sem).wait()
    pltpu.make_async_copy(nib.at[idx_v.at[0]], niv, sem).wait()
    pltpu.sync_copy(rows0, out0.at[pl.ds(w * LANE, LANE)])
    pltpu.sync_copy(rows1, out1.at[pl.ds(w * LANE, LANE)])
    pltpu.sync_copy(niv, out_ni.at[pl.ds(w * LANE, LANE)])


_BN = 2000


def _matmul1_body(h_ref, w1_ref, hw_ref):
  hw_ref[...] = jnp.dot(h_ref[...], w1_ref[...],
                        preferred_element_type=jnp.float32)


def _matmul1(h, w1):
  grid = (N // _BN,)
  return pl.pallas_call(
      _matmul1_body,
      grid=grid,
      in_specs=[
          pl.BlockSpec((_BN, D), lambda i: (i, 0)),
          pl.BlockSpec((D, H), lambda i: (0, 0)),
      ],
      out_specs=pl.BlockSpec((_BN, H), lambda i: (i, 0)),
      out_shape=jax.ShapeDtypeStruct((N, H), jnp.float32),
  )(h, w1)


def _prep_body(hw_ref, dego_ref, degi_ref, t1_ref, no_ref, ni_ref,
               nib_ref):
  do = dego_ref[:, 0:1] + dego_ref[:, 1:2]
  di = degi_ref[:, 0:1] + degi_ref[:, 1:2]
  no = lax.rsqrt(jnp.maximum(do, 1.0))
  ni = lax.rsqrt(jnp.maximum(di, 1.0))
  no_ref[...] = no
  ni_ref[...] = ni
  nib_ref[...] = jnp.broadcast_to(ni, (ni.shape[0], 16))
  t1_ref[...] = hw_ref[...] * no


def _prep(hw, dego, degi):
  grid = (N // _BN,)
  return pl.pallas_call(
      _prep_body,
      grid=grid,
      in_specs=[
          pl.BlockSpec((_BN, H), lambda i: (i, 0)),
          pl.BlockSpec((_BN, 2), lambda i: (i, 0)),
          pl.BlockSpec((_BN, 2), lambda i: (i, 0)),
      ],
      out_specs=[
          pl.BlockSpec((_BN, H), lambda i: (i, 0)),
          pl.BlockSpec((_BN, 1), lambda i: (i, 0)),
          pl.BlockSpec((_BN, 1), lambda i: (i, 0)),
          pl.BlockSpec((_BN, 16), lambda i: (i, 0)),
      ],
      out_shape=[
          jax.ShapeDtypeStruct((N, H), jnp.float32),
          jax.ShapeDtypeStruct((N, 1), jnp.float32),
          jax.ShapeDtypeStruct((N, 1), jnp.float32),
          jax.ShapeDtypeStruct((N, 16), jnp.float32),
      ],
  )(hw, dego, degi)


def _mid_body(p0_ref, p1_ref, ni_ref, no_ref, w2_ref, b1_ref, t2_ref):
  x = jnp.maximum((p0_ref[...] + p1_ref[...]) * ni_ref[...] + b1_ref[...],
                  0.0)
  t2_ref[...] = jnp.dot(x * no_ref[...], w2_ref[...],
                        preferred_element_type=jnp.float32)


def _mid(p0, p1, ni, no, w2, b1):
  grid = (N // _BN,)
  return pl.pallas_call(
      _mid_body,
      grid=grid,
      in_specs=[
          pl.BlockSpec((_BN, H), lambda i: (i, 0)),
          pl.BlockSpec((_BN, H), lambda i: (i, 0)),
          pl.BlockSpec((_BN, 1), lambda i: (i, 0)),
          pl.BlockSpec((_BN, 1), lambda i: (i, 0)),
          pl.BlockSpec((H, H), lambda i: (0, 0)),
          pl.BlockSpec((1, H), lambda i: (0, 0)),
      ],
      out_specs=pl.BlockSpec((_BN, H), lambda i: (i, 0)),
      out_shape=jax.ShapeDtypeStruct((N, H), jnp.float32),
  )(p0, p1, ni, no, w2, b1)


def _head_body(c0_ref, c1_ref, nit_ref, b2_ref, wp1_ref, bp1_ref, wp2_ref,
               bp2_ref, out_ref):
  x2 = (c0_ref[...] + c1_ref[...]) * nit_ref[:, 0:1] + b2_ref[...]
  home = x2[0:B]
  away = x2[B:2 * B]
  z = (jnp.dot(home, wp1_ref[0:H], preferred_element_type=jnp.float32)
       + jnp.dot(away, wp1_ref[H:2 * H], preferred_element_type=jnp.float32)
       + bp1_ref[...])
  z = jnp.maximum(z, 0.0)
  out_ref[...] = jnp.dot(z, wp2_ref[...],
                         preferred_element_type=jnp.float32) + bp2_ref[...]


def _head(c0, c1, nit, b2, wp1, bp1, wp2, bp2):
  return pl.pallas_call(
      _head_body,
      out_shape=jax.ShapeDtypeStruct((B, 2), jnp.float32),
  )(c0, c1, nit, b2, wp1, bp1, wp2, bp2)


def kernel(h, edge_index, team_indices, W1, b1, W2, b2, Wp1, bp1, Wp2, bp2):
  e = edge_index.shape[1]
  rreal = e // LANE
  rpw = _worker_counts(rreal)
  rpad = rpw * NW

  src = edge_index[0].astype(jnp.int32)
  dst = edge_index[1].astype(jnp.int32)
  pad = rpad * LANE - e
  srcr = jnp.concatenate([src, jnp.zeros((pad,), jnp.int32)]).reshape(rpad, LANE)
  dstr = jnp.concatenate([dst, jnp.zeros((pad,), jnp.int32)]).reshape(rpad, LANE)

  zeros1 = jnp.zeros((NACC1,), jnp.float32)
  zeros2 = jnp.zeros((NACC2, H), jnp.float32)

  hw = _matmul1(h, W1)
  degp = _make_deg(rpad, rreal, rpw)(srcr, dstr, zeros1)
  dego = degp[:, 0, :N].T
  degi = degp[:, 1, :N].T

  t1, no, ni, nib = _prep(hw, dego, degi)
  seg = _make_seg(rpad, rreal, rpw)
  p0, p1 = seg(t1, srcr, dstr, zeros2)
  t2 = _mid(p0, p1, ni, no, W2, b1.reshape(1, H))
  tir = team_indices.T.astype(jnp.int32).reshape(2 * B // LANE, 1, LANE)
  c0, c1, nit = _make_seg_gather(rpad, rreal, rpw)(t2, srcr, dstr, zeros2,
                                                   nib, tir)

  return _head(c0, c1, nit, b2.reshape(1, H), Wp1,
               bp1.reshape(1, H), Wp2, bp2.reshape(1, 2))


# speedup vs baseline: 2.5880x; 1.0243x over previous
import functools

import jax
import jax.numpy as jnp
from jax import lax
from jax.experimental import pallas as pl
from jax.experimental.pallas import tpu as pltpu
from jax.experimental.pallas import tpu_sc as plsc

N = 10000
D = 128
H = 64
B = 1024

NC = 2
NS = 16
NW = NC * NS
LANE = 128

NACC1 = 10240
RPT1 = NACC1 // NS
NACC2 = 10240
OUT_RPT = NACC2 // NS

_mesh = plsc.VectorSubcoreMesh(core_axis_name="c", subcore_axis_name="s")
_SC_PARAMS = pltpu.CompilerParams(use_tc_tiling_on_sc=False)


def _worker_counts(num_rows):
  rpw = (num_rows + NW - 1) // NW
  return (rpw + 7) // 8 * 8


def _make_deg(rpad, rreal, rpw):
  @functools.partial(
      pl.kernel,
      mesh=_mesh,
      compiler_params=_SC_PARAMS,
      out_type=jax.ShapeDtypeStruct((NC, 2, NACC1), jnp.float32),
      scratch_types=[
          pltpu.VMEM_SHARED((NACC1,), jnp.float32),
          pltpu.VMEM_SHARED((NACC1,), jnp.float32),
          pltpu.VMEM((LANE,), jnp.float32),
          pltpu.VMEM((LANE,), jnp.float32),
          pltpu.VMEM((rpw, LANE), jnp.int32),
          pltpu.VMEM((rpw, LANE), jnp.int32),
          pltpu.SemaphoreType.DMA,
      ],
  )
  def deg_kernel(srcr, dstr, out, acc_o, acc_i, zbuf1, ones_v, src_v, dst_v,
                 sem):
    c = lax.axis_index("c")
    s = lax.axis_index("s")
    w = s * NC + c
    def zfill(i, carry):
      zbuf1[pl.ds(i * 16, 16)] = jnp.zeros((16,), jnp.float32)
      return carry

    lax.fori_loop(0, 8, zfill, 0)

    def zcopy(m, carry):
      pltpu.sync_copy(zbuf1, acc_o.at[pl.ds(s * RPT1 + m * 128, 128)])
      pltpu.sync_copy(zbuf1, acc_i.at[pl.ds(s * RPT1 + m * 128, 128)])
      return carry

    lax.fori_loop(0, RPT1 // 128, zcopy, 0)

    def fill(i, carry):
      ones_v[pl.ds(i * 16, 16)] = jnp.full((16,), 1.0, jnp.float32)
      return carry

    lax.fori_loop(0, LANE // 16, fill, 0)
    pltpu.sync_copy(srcr.at[pl.ds(w * rpw, rpw)], src_v)
    pltpu.sync_copy(dstr.at[pl.ds(w * rpw, rpw)], dst_v)
    plsc.subcore_barrier()

    cnt = jnp.minimum(rpw, jnp.maximum(0, rreal - w * rpw))

    K = 8

    def step(j, carry):
      pltpu.async_copy(ones_v, acc_o.at[src_v.at[j]], sem, add=True)
      pltpu.async_copy(ones_v, acc_i.at[dst_v.at[j]], sem, add=True)

      @pl.when(j >= K)
      def _():
        pltpu.make_async_copy(ones_v, acc_o.at[src_v.at[j - K]], sem).wait()
        pltpu.make_async_copy(ones_v, acc_i.at[dst_v.at[j - K]], sem).wait()

      return carry

    lax.fori_loop(0, cnt, step, 0)

    def tail(j, carry):
      pltpu.make_async_copy(ones_v, acc_o.at[src_v.at[j]], sem).wait()
      pltpu.make_async_copy(ones_v, acc_i.at[dst_v.at[j]], sem).wait()
      return carry

    lax.fori_loop(jnp.maximum(0, cnt - K), cnt, tail, 0)
    plsc.subcore_barrier()
    pltpu.sync_copy(acc_o.at[pl.ds(s * RPT1, RPT1)],
                    out.at[c, 0, pl.ds(s * RPT1, RPT1)])
    pltpu.sync_copy(acc_i.at[pl.ds(s * RPT1, RPT1)],
                    out.at[c, 1, pl.ds(s * RPT1, RPT1)])

  return deg_kernel


def _make_seg(rpad, rreal, rpw):
  @functools.partial(
      pl.kernel,
      mesh=_mesh,
      compiler_params=_SC_PARAMS,
      out_type=(jax.ShapeDtypeStruct((NACC2, H), jnp.float32),
                jax.ShapeDtypeStruct((NACC2, H), jnp.float32)),
      scratch_types=[
          pltpu.VMEM_SHARED((NACC2, H), jnp.float32),
          pltpu.VMEM((64, H), jnp.float32),
          pltpu.VMEM((LANE, H), jnp.float32),
          pltpu.VMEM((LANE, H), jnp.float32),
          pltpu.VMEM((LANE, H), jnp.float32),
          pltpu.VMEM((LANE, H), jnp.float32),
          pltpu.VMEM((rpw, LANE), jnp.int32),
          pltpu.VMEM((rpw, LANE), jnp.int32),
          pltpu.SemaphoreType.DMA,
          pltpu.SemaphoreType.DMA,
          pltpu.SemaphoreType.DMA,
          pltpu.SemaphoreType.DMA,
          pltpu.SemaphoreType.DMA,
          pltpu.SemaphoreType.DMA,
          pltpu.SemaphoreType.DMA,
          pltpu.SemaphoreType.DMA,
      ],
  )
  def seg_kernel(t, srcr, dstr, out0, out1, acc, zbuf, rows_0, rows_1,
                 rows_2, rows_3, src_v, dst_v, g0, g1, g2, g3, s0, s1, s2,
                 s3):
    c = lax.axis_index("c")
    s = lax.axis_index("s")
    w = s * NC + c
    def zfill(i, carry):
      for kk in range(4):
        zbuf[i, pl.ds(kk * 16, 16)] = jnp.zeros((16,), jnp.float32)
      return carry

    lax.fori_loop(0, 64, zfill, 0)

    def zcopy(m, carry):
      pltpu.sync_copy(zbuf, acc.at[pl.ds(s * OUT_RPT + m * 64, 64)])
      return carry

    lax.fori_loop(0, OUT_RPT // 64, zcopy, 0)
    pltpu.sync_copy(srcr.at[pl.ds(w * rpw, rpw)], src_v)
    pltpu.sync_copy(dstr.at[pl.ds(w * rpw, rpw)], dst_v)
    plsc.subcore_barrier()

    cnt = jnp.minimum(rpw, jnp.maximum(0, rreal - w * rpw))
    bufs = ((rows_0, g0, s0), (rows_1, g1, s1),
            (rows_2, g2, s2), (rows_3, g3, s3))

    for k, (buf, gsem, _) in enumerate(bufs):
      @pl.when(k < cnt)
      def _(k=k, buf=buf, gsem=gsem):
        pltpu.async_copy(t.at[src_v.at[k]], buf, gsem)

    def quad(i, carry):
      j = 4 * i
      for k, (buf, gsem, ssem) in enumerate(bufs):
        pltpu.make_async_copy(t.at[src_v.at[j + k]], buf, gsem).wait()
        pltpu.async_copy(buf, acc.at[dst_v.at[j + k]], ssem, add=True)
      for k, (buf, gsem, ssem) in enumerate(bufs):
        pltpu.make_async_copy(buf, acc.at[dst_v.at[j + k]], ssem).wait()

        @pl.when(j + 4 + k < cnt)
        def _(k=k, buf=buf, gsem=gsem, j=j):
          pltpu.async_copy(t.at[src_v.at[j + 4 + k]], buf, gsem)

      return carry

    lax.fori_loop(0, cnt // 4, quad, 0)
    plsc.subcore_barrier()

    @pl.when(c == 0)
    def _():
      pltpu.sync_copy(acc.at[pl.ds(s * OUT_RPT, OUT_RPT)],
                      out0.at[pl.ds(s * OUT_RPT, OUT_RPT)])

    @pl.when(c == 1)
    def _():
      pltpu.sync_copy(acc.at[pl.ds(s * OUT_RPT, OUT_RPT)],
                      out1.at[pl.ds(s * OUT_RPT, OUT_RPT)])

  return seg_kernel


def _make_seg_gather(rpad, rreal, rpw):
  @functools.partial(
      pl.kernel,
      mesh=_mesh,
      compiler_params=_SC_PARAMS,
      out_type=(jax.ShapeDtypeStruct((2 * B, H), jnp.float32),
                jax.ShapeDtypeStruct((2 * B, H), jnp.float32),
                jax.ShapeDtypeStruct((2 * B, 16), jnp.float32)),
      scratch_types=[
          pltpu.VMEM_SHARED((NACC2, H), jnp.float32),
          pltpu.VMEM((64, H), jnp.float32),
          pltpu.VMEM((LANE, H), jnp.float32),
          pltpu.VMEM((LANE, H), jnp.float32),
          pltpu.VMEM((LANE, H), jnp.float32),
          pltpu.VMEM((LANE, H), jnp.float32),
          pltpu.VMEM((LANE, 16), jnp.float32),
          pltpu.VMEM((1, LANE), jnp.int32),
          pltpu.VMEM((rpw, LANE), jnp.int32),
          pltpu.VMEM((rpw, LANE), jnp.int32),
          pltpu.SemaphoreType.DMA,
          pltpu.SemaphoreType.DMA,
          pltpu.SemaphoreType.DMA,
          pltpu.SemaphoreType.DMA,
          pltpu.SemaphoreType.DMA,
          pltpu.SemaphoreType.DMA,
          pltpu.SemaphoreType.DMA,
          pltpu.SemaphoreType.DMA,
      ],
  )
  def seg_gather_kernel(t, srcr, dstr, nib, tir, c0_out, c1_out,
                        nit_out, acc, zbuf, rows_0, rows_1, rows_2, rows_3,
                        niv, idx_v, src_v, dst_v, g0, g1, g2, g3, s0, s1, s2,
                        s3):
    c = lax.axis_index("c")
    s = lax.axis_index("s")
    w = s * NC + c
    def zfill(i, carry):
      for kk in range(4):
        zbuf[i, pl.ds(kk * 16, 16)] = jnp.zeros((16,), jnp.float32)
      return carry

    lax.fori_loop(0, 64, zfill, 0)

    def zcopy(m, carry):
      pltpu.sync_copy(zbuf, acc.at[pl.ds(s * OUT_RPT + m * 64, 64)])
      return carry

    lax.fori_loop(0, OUT_RPT // 64, zcopy, 0)
    pltpu.sync_copy(srcr.at[pl.ds(w * rpw, rpw)], src_v)
    pltpu.sync_copy(dstr.at[pl.ds(w * rpw, rpw)], dst_v)
    plsc.subcore_barrier()

    cnt = jnp.minimum(rpw, jnp.maximum(0, rreal - w * rpw))
    bufs = ((rows_0, g0, s0), (rows_1, g1, s1),
            (rows_2, g2, s2), (rows_3, g3, s3))

    for k, (buf, gsem, _) in enumerate(bufs):
      @pl.when(k < cnt)
      def _(k=k, buf=buf, gsem=gsem):
        pltpu.async_copy(t.at[src_v.at[k]], buf, gsem)

    def quad(i, carry):
      j = 4 * i
      for k, (buf, gsem, ssem) in enumerate(bufs):
        pltpu.make_async_copy(t.at[src_v.at[j + k]], buf, gsem).wait()
        pltpu.async_copy(buf, acc.at[dst_v.at[j + k]], ssem, add=True)
      for k, (buf, gsem, ssem) in enumerate(bufs):
        pltpu.make_async_copy(buf, acc.at[dst_v.at[j + k]], ssem).wait()

        @pl.when(j + 4 + k < cnt)
        def _(k=k, buf=buf, gsem=gsem, j=j):
          pltpu.async_copy(t.at[src_v.at[j + 4 + k]], buf, gsem)

      return carry

    lax.fori_loop(0, cnt // 4, quad, 0)
    plsc.subcore_barrier()

    pltpu.sync_copy(tir.at[s], idx_v)
    pltpu.async_copy(acc.at[idx_v.at[0]], rows_0, g0).wait()

    @pl.when(c == 0)
    def _():
      pltpu.sync_copy(rows_0, c0_out.at[pl.ds(s * LANE, LANE)])
      pltpu.async_copy(nib.at[idx_v.at[0]], niv, g1).wait()
      pltpu.sync_copy(niv, nit_out.at[pl.ds(s * LANE, LANE)])

    @pl.when(c == 1)
    def _():
      pltpu.sync_copy(rows_0, c1_out.at[pl.ds(s * LANE, LANE)])

  return seg_gather_kernel


@functools.partial(
    pl.kernel,
    mesh=_mesh,
    compiler_params=_SC_PARAMS,
    out_type=(jax.ShapeDtypeStruct((2 * B, H), jnp.float32),
              jax.ShapeDtypeStruct((2 * B, H), jnp.float32),
              jax.ShapeDtypeStruct((2 * B, 16), jnp.float32)),
    scratch_types=[
        pltpu.VMEM((LANE, H), jnp.float32),
        pltpu.VMEM((LANE, H), jnp.float32),
        pltpu.VMEM((LANE, 16), jnp.float32),
        pltpu.VMEM((1, LANE), jnp.int32),
        pltpu.SemaphoreType.DMA,
    ],
)
def _team_gather(q0, q1, nib, tir, out0, out1, out_ni, rows0, rows1, niv,
                 idx_v, sem):
  c = lax.axis_index("c")
  s = lax.axis_index("s")
  w = s * NC + c
  nrows = (2 * B) // LANE

  @pl.when(w < nrows)
  def _():
    pltpu.sync_copy(tir.at[w], idx_v)
    pltpu.async_copy(q0.at[idx_v.at[0]], rows0, sem)
    pltpu.async_copy(q1.at[idx_v.at[0]], rows1, sem)
    pltpu.async_copy(nib.at[idx_v.at[0]], niv, sem)
    pltpu.make_async_copy(q0.at[idx_v.at[0]], rows0, sem).wait()
    pltpu.make_async_copy(q1.at[idx_v.at[0]], rows1, sem).wait()
    pltpu.make_async_copy(nib.at[idx_v.at[0]], niv, sem).wait()
    pltpu.sync_copy(rows0, out0.at[pl.ds(w * LANE, LANE)])
    pltpu.sync_copy(rows1, out1.at[pl.ds(w * LANE, LANE)])
    pltpu.sync_copy(niv, out_ni.at[pl.ds(w * LANE, LANE)])


_BN = 2000


def _prep_body(h_ref, dego_ref, degi_ref, w1_ref, t1_ref, no_ref, ni_ref,
               nib_ref):
  do = dego_ref[:, 0:1] + dego_ref[:, 1:2]
  di = degi_ref[:, 0:1] + degi_ref[:, 1:2]
  no = lax.rsqrt(jnp.maximum(do, 1.0))
  ni = lax.rsqrt(jnp.maximum(di, 1.0))
  no_ref[...] = no
  ni_ref[...] = ni
  nib_ref[...] = jnp.broadcast_to(ni, (ni.shape[0], 16))
  t1_ref[...] = jnp.dot(h_ref[...] * no, w1_ref[...],
                        preferred_element_type=jnp.float32)


def _prep(h, dego, degi, w1):
  grid = (N // _BN,)
  return pl.pallas_call(
      _prep_body,
      grid=grid,
      in_specs=[
          pl.BlockSpec((_BN, D), lambda i: (i, 0)),
          pl.BlockSpec((_BN, 2), lambda i: (i, 0)),
          pl.BlockSpec((_BN, 2), lambda i: (i, 0)),
          pl.BlockSpec((D, H), lambda i: (0, 0)),
      ],
      out_specs=[
          pl.BlockSpec((_BN, H), lambda i: (i, 0)),
          pl.BlockSpec((_BN, 1), lambda i: (i, 0)),
          pl.BlockSpec((_BN, 1), lambda i: (i, 0)),
          pl.BlockSpec((_BN, 16), lambda i: (i, 0)),
      ],
      out_shape=[
          jax.ShapeDtypeStruct((N, H), jnp.float32),
          jax.ShapeDtypeStruct((N, 1), jnp.float32),
          jax.ShapeDtypeStruct((N, 1), jnp.float32),
          jax.ShapeDtypeStruct((N, 16), jnp.float32),
      ],
  )(h, dego, degi, w1)


def _mid_body(p0_ref, p1_ref, ni_ref, no_ref, w2_ref, b1_ref, t2_ref):
  x = jnp.maximum((p0_ref[...] + p1_ref[...]) * ni_ref[...] + b1_ref[...],
                  0.0)
  t2_ref[...] = jnp.dot(x * no_ref[...], w2_ref[...],
                        preferred_element_type=jnp.float32)


def _mid(p0, p1, ni, no, w2, b1):
  grid = (N // _BN,)
  return pl.pallas_call(
      _mid_body,
      grid=grid,
      in_specs=[
          pl.BlockSpec((_BN, H), lambda i: (i, 0)),
          pl.BlockSpec((_BN, H), lambda i: (i, 0)),
          pl.BlockSpec((_BN, 1), lambda i: (i, 0)),
          pl.BlockSpec((_BN, 1), lambda i: (i, 0)),
          pl.BlockSpec((H, H), lambda i: (0, 0)),
          pl.BlockSpec((1, H), lambda i: (0, 0)),
      ],
      out_specs=pl.BlockSpec((_BN, H), lambda i: (i, 0)),
      out_shape=jax.ShapeDtypeStruct((N, H), jnp.float32),
  )(p0, p1, ni, no, w2, b1)


def _head_body(c0_ref, c1_ref, nit_ref, b2_ref, wp1_ref, bp1_ref, wp2_ref,
               bp2_ref, out_ref):
  x2 = (c0_ref[...] + c1_ref[...]) * nit_ref[:, 0:1] + b2_ref[...]
  home = x2[0:B]
  away = x2[B:2 * B]
  z = (jnp.dot(home, wp1_ref[0:H], preferred_element_type=jnp.float32)
       + jnp.dot(away, wp1_ref[H:2 * H], preferred_element_type=jnp.float32)
       + bp1_ref[...])
  z = jnp.maximum(z, 0.0)
  out_ref[...] = jnp.dot(z, wp2_ref[...],
                         preferred_element_type=jnp.float32) + bp2_ref[...]


def _head(c0, c1, nit, b2, wp1, bp1, wp2, bp2):
  return pl.pallas_call(
      _head_body,
      out_shape=jax.ShapeDtypeStruct((B, 2), jnp.float32),
  )(c0, c1, nit, b2, wp1, bp1, wp2, bp2)


def kernel(h, edge_index, team_indices, W1, b1, W2, b2, Wp1, bp1, Wp2, bp2):
  e = edge_index.shape[1]
  rreal = e // LANE
  rpw = _worker_counts(rreal)
  rpad = rpw * NW

  src = edge_index[0].astype(jnp.int32)
  dst = edge_index[1].astype(jnp.int32)
  pad = rpad * LANE - e
  srcr = jnp.concatenate([src, jnp.zeros((pad,), jnp.int32)]).reshape(rpad, LANE)
  dstr = jnp.concatenate([dst, jnp.zeros((pad,), jnp.int32)]).reshape(rpad, LANE)

  degp = _make_deg(rpad, rreal, rpw)(srcr, dstr)
  dego = degp[:, 0, :N].T
  degi = degp[:, 1, :N].T

  t1, no, ni, nib = _prep(h, dego, degi, W1)
  seg = _make_seg(rpad, rreal, rpw)
  p0, p1 = seg(t1, srcr, dstr)
  t2 = _mid(p0, p1, ni, no, W2, b1.reshape(1, H))
  tir = team_indices.T.astype(jnp.int32).reshape(2 * B // LANE, 1, LANE)
  c0, c1, nit = _make_seg_gather(rpad, rreal, rpw)(t2, srcr, dstr,
                                                   nib, tir)

  return _head(c0, c1, nit, b2.reshape(1, H), Wp1,
               bp1.reshape(1, H), Wp2, bp2.reshape(1, 2))


# speedup vs baseline: 2.6169x; 1.0112x over previous
import functools

import jax
import jax.numpy as jnp
from jax import lax
from jax.experimental import pallas as pl
from jax.experimental.pallas import tpu as pltpu
from jax.experimental.pallas import tpu_sc as plsc

N = 10000
D = 128
H = 64
B = 1024

NC = 2
NS = 16
NW = NC * NS
LANE = 128

NACC1 = 10240
RPT1 = NACC1 // NS
NACC2 = 10240
OUT_RPT = NACC2 // NS

_mesh = plsc.VectorSubcoreMesh(core_axis_name="c", subcore_axis_name="s")
_SC_PARAMS = pltpu.CompilerParams(use_tc_tiling_on_sc=False)


def _worker_counts(num_rows):
  rpw = (num_rows + NW - 1) // NW
  return (rpw + 7) // 8 * 8


def _make_deg(rpad, rreal, rpw):
  @functools.partial(
      pl.kernel,
      mesh=_mesh,
      compiler_params=_SC_PARAMS,
      out_type=jax.ShapeDtypeStruct((NC, 2, NACC1), jnp.float32),
      scratch_types=[
          pltpu.VMEM_SHARED((NACC1,), jnp.float32),
          pltpu.VMEM_SHARED((NACC1,), jnp.float32),
          pltpu.VMEM((LANE,), jnp.float32),
          pltpu.VMEM((LANE,), jnp.float32),
          pltpu.VMEM((rpw, LANE), jnp.int32),
          pltpu.VMEM((rpw, LANE), jnp.int32),
          pltpu.SemaphoreType.DMA,
      ],
  )
  def deg_kernel(srcr, dstr, out, acc_o, acc_i, zbuf1, ones_v, src_v, dst_v,
                 sem):
    c = lax.axis_index("c")
    s = lax.axis_index("s")
    w = s * NC + c
    def zfill(i, carry):
      zbuf1[pl.ds(i * 16, 16)] = jnp.zeros((16,), jnp.float32)
      return carry

    lax.fori_loop(0, 8, zfill, 0)

    def zcopy(m, carry):
      pltpu.sync_copy(zbuf1, acc_o.at[pl.ds(s * RPT1 + m * 128, 128)])
      pltpu.sync_copy(zbuf1, acc_i.at[pl.ds(s * RPT1 + m * 128, 128)])
      return carry

    lax.fori_loop(0, RPT1 // 128, zcopy, 0)

    def fill(i, carry):
      ones_v[pl.ds(i * 16, 16)] = jnp.full((16,), 1.0, jnp.float32)
      return carry

    lax.fori_loop(0, LANE // 16, fill, 0)
    pltpu.sync_copy(srcr.at[pl.ds(w * rpw, rpw)], src_v)
    pltpu.sync_copy(dstr.at[pl.ds(w * rpw, rpw)], dst_v)
    plsc.subcore_barrier()

    cnt = jnp.minimum(rpw, jnp.maximum(0, rreal - w * rpw))

    K = 8

    def step(j, carry):
      pltpu.async_copy(ones_v, acc_o.at[src_v.at[j]], sem, add=True)
      pltpu.async_copy(ones_v, acc_i.at[dst_v.at[j]], sem, add=True)

      @pl.when(j >= K)
      def _():
        pltpu.make_async_copy(ones_v, acc_o.at[src_v.at[j - K]], sem).wait()
        pltpu.make_async_copy(ones_v, acc_i.at[dst_v.at[j - K]], sem).wait()

      return carry

    lax.fori_loop(0, cnt, step, 0)

    def tail(j, carry):
      pltpu.make_async_copy(ones_v, acc_o.at[src_v.at[j]], sem).wait()
      pltpu.make_async_copy(ones_v, acc_i.at[dst_v.at[j]], sem).wait()
      return carry

    lax.fori_loop(jnp.maximum(0, cnt - K), cnt, tail, 0)
    plsc.subcore_barrier()
    pltpu.sync_copy(acc_o.at[pl.ds(s * RPT1, RPT1)],
                    out.at[c, 0, pl.ds(s * RPT1, RPT1)])
    pltpu.sync_copy(acc_i.at[pl.ds(s * RPT1, RPT1)],
                    out.at[c, 1, pl.ds(s * RPT1, RPT1)])

  return deg_kernel


def _make_seg(rpad, rreal, rpw):
  @functools.partial(
      pl.kernel,
      mesh=_mesh,
      compiler_params=_SC_PARAMS,
      out_type=(jax.ShapeDtypeStruct((NACC2, H), jnp.float32),
                jax.ShapeDtypeStruct((NACC2, H), jnp.float32)),
      scratch_types=[
          pltpu.VMEM_SHARED((NACC2, H), jnp.float32),
          pltpu.VMEM((64, H), jnp.float32),
          pltpu.VMEM((LANE, H), jnp.float32),
          pltpu.VMEM((LANE, H), jnp.float32),
          pltpu.VMEM((LANE, H), jnp.float32),
          pltpu.VMEM((LANE, H), jnp.float32),
          pltpu.VMEM((rpw, LANE), jnp.int32),
          pltpu.VMEM((rpw, LANE), jnp.int32),
          pltpu.SemaphoreType.DMA,
          pltpu.SemaphoreType.DMA,
          pltpu.SemaphoreType.DMA,
          pltpu.SemaphoreType.DMA,
          pltpu.SemaphoreType.DMA,
          pltpu.SemaphoreType.DMA,
          pltpu.SemaphoreType.DMA,
          pltpu.SemaphoreType.DMA,
      ],
  )
  def seg_kernel(t, srcr, dstr, out0, out1, acc, zbuf, rows_0, rows_1,
                 rows_2, rows_3, src_v, dst_v, g0, g1, g2, g3, s0, s1, s2,
                 s3):
    c = lax.axis_index("c")
    s = lax.axis_index("s")
    w = s * NC + c
    def zfill(i, carry):
      for kk in range(4):
        zbuf[i, pl.ds(kk * 16, 16)] = jnp.zeros((16,), jnp.float32)
      return carry

    lax.fori_loop(0, 64, zfill, 0)

    def zcopy(m, carry):
      pltpu.sync_copy(zbuf, acc.at[pl.ds(s * OUT_RPT + m * 64, 64)])
      return carry

    lax.fori_loop(0, OUT_RPT // 64, zcopy, 0)
    pltpu.sync_copy(srcr.at[pl.ds(w * rpw, rpw)], src_v)
    pltpu.sync_copy(dstr.at[pl.ds(w * rpw, rpw)], dst_v)
    plsc.subcore_barrier()

    cnt = jnp.minimum(rpw, jnp.maximum(0, rreal - w * rpw))
    bufs = ((rows_0, g0, s0), (rows_1, g1, s1),
            (rows_2, g2, s2), (rows_3, g3, s3))

    for k, (buf, gsem, _) in enumerate(bufs):
      @pl.when(k < cnt)
      def _(k=k, buf=buf, gsem=gsem):
        pltpu.async_copy(t.at[src_v.at[k]], buf, gsem)

    def quad(i, carry):
      j = 4 * i
      for k, (buf, gsem, ssem) in enumerate(bufs):
        pltpu.make_async_copy(t.at[src_v.at[j + k]], buf, gsem).wait()
        pltpu.async_copy(buf, acc.at[dst_v.at[j + k]], ssem, add=True)
      for k, (buf, gsem, ssem) in enumerate(bufs):
        pltpu.make_async_copy(buf, acc.at[dst_v.at[j + k]], ssem).wait()

        @pl.when(j + 4 + k < cnt)
        def _(k=k, buf=buf, gsem=gsem, j=j):
          pltpu.async_copy(t.at[src_v.at[j + 4 + k]], buf, gsem)

      return carry

    lax.fori_loop(0, cnt // 4, quad, 0)
    plsc.subcore_barrier()

    @pl.when(c == 0)
    def _():
      pltpu.sync_copy(acc.at[pl.ds(s * OUT_RPT, OUT_RPT)],
                      out0.at[pl.ds(s * OUT_RPT, OUT_RPT)])

    @pl.when(c == 1)
    def _():
      pltpu.sync_copy(acc.at[pl.ds(s * OUT_RPT, OUT_RPT)],
                      out1.at[pl.ds(s * OUT_RPT, OUT_RPT)])

  return seg_kernel


def _make_seg_gather(rpad, rreal, rpw):
  @functools.partial(
      pl.kernel,
      mesh=_mesh,
      compiler_params=_SC_PARAMS,
      out_type=(jax.ShapeDtypeStruct((2 * B, H), jnp.float32),
                jax.ShapeDtypeStruct((2 * B, H), jnp.float32),
                jax.ShapeDtypeStruct((2 * B, 16), jnp.float32)),
      scratch_types=[
          pltpu.VMEM_SHARED((NACC2, H), jnp.float32),
          pltpu.VMEM((64, H), jnp.float32),
          pltpu.VMEM((LANE, H), jnp.float32),
          pltpu.VMEM((LANE, H), jnp.float32),
          pltpu.VMEM((LANE, H), jnp.float32),
          pltpu.VMEM((LANE, H), jnp.float32),
          pltpu.VMEM((LANE, 16), jnp.float32),
          pltpu.VMEM((1, LANE), jnp.int32),
          pltpu.VMEM((rpw, LANE), jnp.int32),
          pltpu.VMEM((rpw, LANE), jnp.int32),
          pltpu.SemaphoreType.DMA,
          pltpu.SemaphoreType.DMA,
          pltpu.SemaphoreType.DMA,
          pltpu.SemaphoreType.DMA,
          pltpu.SemaphoreType.DMA,
          pltpu.SemaphoreType.DMA,
          pltpu.SemaphoreType.DMA,
          pltpu.SemaphoreType.DMA,
      ],
  )
  def seg_gather_kernel(t, srcr, dstr, nib, tir, c0_out, c1_out,
                        nit_out, acc, zbuf, rows_0, rows_1, rows_2, rows_3,
                        niv, idx_v, src_v, dst_v, g0, g1, g2, g3, s0, s1, s2,
                        s3):
    c = lax.axis_index("c")
    s = lax.axis_index("s")
    w = s * NC + c
    def zfill(i, carry):
      for kk in range(4):
        zbuf[i, pl.ds(kk * 16, 16)] = jnp.zeros((16,), jnp.float32)
      return carry

    lax.fori_loop(0, 64, zfill, 0)

    def zcopy(m, carry):
      pltpu.sync_copy(zbuf, acc.at[pl.ds(s * OUT_RPT + m * 64, 64)])
      return carry

    lax.fori_loop(0, OUT_RPT // 64, zcopy, 0)
    pltpu.sync_copy(srcr.at[pl.ds(w * rpw, rpw)], src_v)
    pltpu.sync_copy(dstr.at[pl.ds(w * rpw, rpw)], dst_v)
    plsc.subcore_barrier()

    cnt = jnp.minimum(rpw, jnp.maximum(0, rreal - w * rpw))
    bufs = ((rows_0, g0, s0), (rows_1, g1, s1),
            (rows_2, g2, s2), (rows_3, g3, s3))

    for k, (buf, gsem, _) in enumerate(bufs):
      @pl.when(k < cnt)
      def _(k=k, buf=buf, gsem=gsem):
        pltpu.async_copy(t.at[src_v.at[k]], buf, gsem)

    def quad(i, carry):
      j = 4 * i
      for k, (buf, gsem, ssem) in enumerate(bufs):
        pltpu.make_async_copy(t.at[src_v.at[j + k]], buf, gsem).wait()
        pltpu.async_copy(buf, acc.at[dst_v.at[j + k]], ssem, add=True)
      for k, (buf, gsem, ssem) in enumerate(bufs):
        pltpu.make_async_copy(buf, acc.at[dst_v.at[j + k]], ssem).wait()

        @pl.when(j + 4 + k < cnt)
        def _(k=k, buf=buf, gsem=gsem, j=j):
          pltpu.async_copy(t.at[src_v.at[j + 4 + k]], buf, gsem)

      return carry

    lax.fori_loop(0, cnt // 4, quad, 0)
    plsc.subcore_barrier()

    pltpu.sync_copy(tir.at[s], idx_v)
    pltpu.async_copy(acc.at[idx_v.at[0]], rows_0, g0).wait()

    @pl.when(c == 0)
    def _():
      pltpu.sync_copy(rows_0, c0_out.at[pl.ds(s * LANE, LANE)])
      pltpu.async_copy(nib.at[idx_v.at[0]], niv, g1).wait()
      pltpu.sync_copy(niv, nit_out.at[pl.ds(s * LANE, LANE)])

    @pl.when(c == 1)
    def _():
      pltpu.sync_copy(rows_0, c1_out.at[pl.ds(s * LANE, LANE)])

  return seg_gather_kernel


_BN = 2000


def _prep_body(h_ref, dego_ref, degi_ref, w1_ref, t1_ref, no_ref, ni_ref,
               nib_ref):
  do = dego_ref[:, 0:1] + dego_ref[:, 1:2]
  di = degi_ref[:, 0:1] + degi_ref[:, 1:2]
  no = lax.rsqrt(jnp.maximum(do, 1.0))
  ni = lax.rsqrt(jnp.maximum(di, 1.0))
  no_ref[...] = no
  ni_ref[...] = ni
  nib_ref[...] = jnp.broadcast_to(ni, (ni.shape[0], 16))
  t1_ref[...] = jnp.dot(h_ref[...] * no, w1_ref[...],
                        preferred_element_type=jnp.float32)


def _prep(h, dego, degi, w1):
  grid = (N // _BN,)
  return pl.pallas_call(
      _prep_body,
      grid=grid,
      in_specs=[
          pl.BlockSpec((_BN, D), lambda i: (i, 0)),
          pl.BlockSpec((_BN, 2), lambda i: (i, 0)),
          pl.BlockSpec((_BN, 2), lambda i: (i, 0)),
          pl.BlockSpec((D, H), lambda i: (0, 0)),
      ],
      out_specs=[
          pl.BlockSpec((_BN, H), lambda i: (i, 0)),
          pl.BlockSpec((_BN, 1), lambda i: (i, 0)),
          pl.BlockSpec((_BN, 1), lambda i: (i, 0)),
          pl.BlockSpec((_BN, 16), lambda i: (i, 0)),
      ],
      out_shape=[
          jax.ShapeDtypeStruct((N, H), jnp.float32),
          jax.ShapeDtypeStruct((N, 1), jnp.float32),
          jax.ShapeDtypeStruct((N, 1), jnp.float32),
          jax.ShapeDtypeStruct((N, 16), jnp.float32),
      ],
  )(h, dego, degi, w1)


def _mid_body(p0_ref, p1_ref, ni_ref, no_ref, w2_ref, b1_ref, t2_ref):
  x = jnp.maximum((p0_ref[...] + p1_ref[...]) * ni_ref[...] + b1_ref[...],
                  0.0)
  t2_ref[...] = jnp.dot(x * no_ref[...], w2_ref[...],
                        preferred_element_type=jnp.float32)


def _mid(p0, p1, ni, no, w2, b1):
  grid = (N // _BN,)
  return pl.pallas_call(
      _mid_body,
      grid=grid,
      in_specs=[
          pl.BlockSpec((_BN, H), lambda i: (i, 0)),
          pl.BlockSpec((_BN, H), lambda i: (i, 0)),
          pl.BlockSpec((_BN, 1), lambda i: (i, 0)),
          pl.BlockSpec((_BN, 1), lambda i: (i, 0)),
          pl.BlockSpec((H, H), lambda i: (0, 0)),
          pl.BlockSpec((1, H), lambda i: (0, 0)),
      ],
      out_specs=pl.BlockSpec((_BN, H), lambda i: (i, 0)),
      out_shape=jax.ShapeDtypeStruct((N, H), jnp.float32),
  )(p0, p1, ni, no, w2, b1)


def _head_body(c0_ref, c1_ref, nit_ref, b2_ref, wp1_ref, bp1_ref, wp2_ref,
               bp2_ref, out_ref):
  x2 = (c0_ref[...] + c1_ref[...]) * nit_ref[:, 0:1] + b2_ref[...]
  home = x2[0:B]
  away = x2[B:2 * B]
  z = (jnp.dot(home, wp1_ref[0:H], preferred_element_type=jnp.float32)
       + jnp.dot(away, wp1_ref[H:2 * H], preferred_element_type=jnp.float32)
       + bp1_ref[...])
  z = jnp.maximum(z, 0.0)
  out_ref[...] = jnp.dot(z, wp2_ref[...],
                         preferred_element_type=jnp.float32) + bp2_ref[...]


def _head(c0, c1, nit, b2, wp1, bp1, wp2, bp2):
  return pl.pallas_call(
      _head_body,
      out_shape=jax.ShapeDtypeStruct((B, 2), jnp.float32),
  )(c0, c1, nit, b2, wp1, bp1, wp2, bp2)


def kernel(h, edge_index, team_indices, W1, b1, W2, b2, Wp1, bp1, Wp2, bp2):
  e = edge_index.shape[1]
  rreal = e // LANE
  rpw = _worker_counts(rreal)
  rpad = rpw * NW

  src = edge_index[0].astype(jnp.int32)
  dst = edge_index[1].astype(jnp.int32)
  pad = rpad * LANE - e
  srcr = jnp.concatenate([src, jnp.zeros((pad,), jnp.int32)]).reshape(rpad, LANE)
  dstr = jnp.concatenate([dst, jnp.zeros((pad,), jnp.int32)]).reshape(rpad, LANE)

  degp = _make_deg(rpad, rreal, rpw)(srcr, dstr)
  dego = degp[:, 0, :N].T
  degi = degp[:, 1, :N].T

  t1, no, ni, nib = _prep(h, dego, degi, W1)
  seg = _make_seg(rpad, rreal, rpw)
  p0, p1 = seg(t1, srcr, dstr)
  t2 = _mid(p0, p1, ni, no, W2, b1.reshape(1, H))
  tir = team_indices.T.astype(jnp.int32).reshape(2 * B // LANE, 1, LANE)
  c0, c1, nit = _make_seg_gather(rpad, rreal, rpw)(t2, srcr, dstr,
                                                   nib, tir)

  return _head(c0, c1, nit, b2.reshape(1, H), Wp1,
               bp1.reshape(1, H), Wp2, bp2.reshape(1, 2))
